# baseline (speedup 1.0000x reference)
import functools

import jax
import jax.numpy as jnp
from jax import lax
from jax.experimental import pallas as pl
from jax.experimental.pallas import tpu as pltpu
from jax.experimental.pallas import tpu_sc as plsc

_N = 10000
_E = 320000
_D = 128
_H = 128
_NC = 2
_NS = 16
_NW = _NC * _NS
_EPT = _E // _NW
_K = 128
_CH = 79
_CHS = 80
_EPAD = _NW * _CH * _K - _E
_NPAD = 80 * 128
_RPT = _NPAD // _NS
_TPAD = _N + 16

_mesh = plsc.VectorSubcoreMesh(core_axis_name="c", subcore_axis_name="s")
_sc_params = pltpu.CompilerParams(needs_layout_passes=False)


@functools.partial(
    pl.kernel,
    out_type=jax.ShapeDtypeStruct((_NC, _NPAD, _H), jnp.float32),
    mesh=_mesh,
    scratch_types=[
        pltpu.VMEM((_CHS, _K), jnp.int32),
        pltpu.VMEM((2, _K), jnp.int32),
        pltpu.VMEM((_K, _H), jnp.float32),
        pltpu.VMEM((_K, _H), jnp.float32),
        pltpu.VMEM_SHARED((_NPAD, _H), jnp.float32),
        pltpu.SemaphoreType.DMA,
        pltpu.SemaphoreType.DMA,
        pltpu.SemaphoreType.DMA,
        pltpu.SemaphoreType.DMA,
    ],
    compiler_params=_sc_params,
)
def _seg_sum_sc(table_hbm, src_hbm, dst_hbm, out_hbm, srcv, dbuf, rb0, rb1,
                acc, gsem0, gsem1, dsem0, dsem1):
    cid = lax.axis_index("c")
    sid = lax.axis_index("s")
    wid = sid * _NC + cid

    pltpu.sync_copy(src_hbm.at[wid], srcv)
    pltpu.async_copy(table_hbm.at[srcv.at[0]], rb0, gsem0)
    pltpu.async_copy(dst_hbm.at[wid, 0], dbuf.at[0], dsem0)

    zeros16 = jnp.zeros((16,), jnp.float32)

    def _zfill(i, carry):
        for u in range(8):
            rb1[i, pl.ds(u * 16, 16)] = zeros16
        return carry

    lax.fori_loop(0, _K, _zfill, 0)
    base = sid * _RPT
    for r in range(_RPT // _K):
        pltpu.async_copy(rb1, acc.at[pl.ds(base + r * _K, _K)], gsem1)
    for r in range(_RPT // _K):
        pltpu.make_async_copy(rb1, acc.at[pl.ds(base + r * _K, _K)],
                              gsem1).wait()
    plsc.subcore_barrier()

    def _body(jj, carry):
        j0 = jj * 2
        pltpu.make_async_copy(table_hbm.at[srcv.at[j0]], rb0, gsem0).wait()
        pltpu.make_async_copy(dst_hbm.at[wid, j0], dbuf.at[0], dsem0).wait()
        pltpu.async_copy(table_hbm.at[srcv.at[j0 + 1]], rb1, gsem1)
        pltpu.async_copy(dst_hbm.at[wid, j0 + 1], dbuf.at[1], dsem1)
        pltpu.sync_copy(rb0, acc.at[dbuf.at[0]], add=True)
        pltpu.make_async_copy(table_hbm.at[srcv.at[j0 + 1]], rb1, gsem1).wait()
        pltpu.make_async_copy(dst_hbm.at[wid, j0 + 1], dbuf.at[1], dsem1).wait()

        @pl.when(jj < _CH // 2 - 1)
        def _():
            pltpu.async_copy(table_hbm.at[srcv.at[j0 + 2]], rb0, gsem0)
            pltpu.async_copy(dst_hbm.at[wid, j0 + 2], dbuf.at[0], dsem0)

        pltpu.sync_copy(rb1, acc.at[dbuf.at[1]], add=True)
        return carry

    lax.fori_loop(0, _CH // 2, _body, 0)
    if _CH % 2:
        pltpu.sync_copy(dst_hbm.at[wid, _CH - 1], dbuf.at[0])
        pltpu.sync_copy(table_hbm.at[srcv.at[_CH - 1]], rb0)
        pltpu.sync_copy(rb0, acc.at[dbuf.at[0]], add=True)

    plsc.subcore_barrier()
    pltpu.sync_copy(acc.at[pl.ds(sid * _RPT, _RPT)],
                    out_hbm.at[cid, pl.ds(sid * _RPT, _RPT)])


@functools.partial(
    pl.kernel,
    out_type=(jax.ShapeDtypeStruct((_NC, 80, 128), jnp.float32),
              jax.ShapeDtypeStruct((_NC, 80, 128), jnp.float32)),
    mesh=_mesh,
    scratch_types=[
        pltpu.VMEM((_EPT,), jnp.int32),
        pltpu.VMEM((_NPAD,), jnp.float32),
        pltpu.VMEM((_NPAD,), jnp.float32),
        pltpu.VMEM((80, 128), jnp.float32),
        pltpu.VMEM((80, 128), jnp.float32),
        pltpu.VMEM((1, 80), jnp.int32),
        pltpu.VMEM_SHARED((80, 128), jnp.float32),
        pltpu.VMEM_SHARED((80, 128), jnp.float32),
    ],
    compiler_params=_sc_params,
)
def _degrees_sc(src_hbm, dst_hbm, outs_hbm, outd_hbm,
                idxv, h1s, h1d, hs, hd, rows_i, shs, shd):
    cid = lax.axis_index("c")
    sid = lax.axis_index("s")
    wid = sid * _NC + cid

    zeros16 = jnp.zeros((16,), jnp.float32)
    ones16 = jnp.ones((16,), jnp.float32)

    def _zfill(i, carry):
        hs[i >> 3, pl.ds((i & 7) * 16, 16)] = zeros16
        hd[i >> 3, pl.ds((i & 7) * 16, 16)] = zeros16
        h1s[pl.ds(i * 16, 16)] = zeros16
        h1d[pl.ds(i * 16, 16)] = zeros16
        return carry

    lax.fori_loop(0, 80 * 8, _zfill, 0)
    for r in range(5):
        rows_i[0, pl.ds(r * 16, 16)] = lax.iota(jnp.int32, 16) + r * 16

    @pl.when(sid < 10)
    def _():
        pltpu.sync_copy(hs.at[pl.ds(sid * 8, 8)], shs.at[pl.ds(sid * 8, 8)])
        pltpu.sync_copy(hd.at[pl.ds(sid * 8, 8)], shd.at[pl.ds(sid * 8, 8)])

    plsc.subcore_barrier()

    pltpu.sync_copy(src_hbm.at[pl.ds(wid * _EPT, _EPT)], idxv)

    def _hist_s(i, carry):
        for u in range(5):
            v = idxv[pl.ds(i * 80 + u * 16, 16)]
            plsc.addupdate_scatter(h1s, [v], ones16)
        return carry

    lax.fori_loop(0, _EPT // 80, _hist_s, 0)

    pltpu.sync_copy(dst_hbm.at[pl.ds(wid * _EPT, _EPT)], idxv)

    def _hist_d(i, carry):
        for u in range(5):
            v = idxv[pl.ds(i * 80 + u * 16, 16)]
            plsc.addupdate_scatter(h1d, [v], ones16)
        return carry

    lax.fori_loop(0, _EPT // 80, _hist_d, 0)

    def _repack(i, carry):
        hs[i >> 3, pl.ds((i & 7) * 16, 16)] = h1s[pl.ds(i * 16, 16)]
        hd[i >> 3, pl.ds((i & 7) * 16, 16)] = h1d[pl.ds(i * 16, 16)]
        return carry

    lax.fori_loop(0, 80 * 8, _repack, 0)

    pltpu.sync_copy(hs, shs.at[rows_i.at[0]], add=True)
    pltpu.sync_copy(hd, shd.at[rows_i.at[0]], add=True)
    plsc.subcore_barrier()

    @pl.when(sid == 0)
    def _():
        pltpu.sync_copy(shs, outs_hbm.at[cid])
        pltpu.sync_copy(shd, outd_hbm.at[cid])


def _enc_body(feat, w1, b1, w2, b2, w3, b3, h_out):
    x = feat[...]
    h = jnp.maximum(jnp.dot(x, w1[...]) + b1[...], 0.0)
    h = jnp.maximum(jnp.dot(h, w2[...]) + b2[...], 0.0)
    h_out[...] = jnp.dot(h, w3[...]) + b3[...]


def _norm_body(henc, odp, idp, hs_out, ns_out, nd_out, inv_out):
    od = jnp.maximum(odp[0] + odp[1], 1.0)
    indeg = jnp.maximum(idp[0] + idp[1], 1.0)
    ns = lax.rsqrt(od)
    ns_out[...] = ns
    nd_out[...] = lax.rsqrt(indeg)
    inv_out[...] = 1.0 / indeg
    hs_out[:_N] = henc[...] * ns
    hs_out[_N:] = jnp.zeros((_TPAD - _N, _H), jnp.float32)


def _att_body(h_ref, nbp, inv, ns, w1h, w1n, b1, w2, b2, h2_out, hs_out):
    h = h_ref[:_N]
    nb = (nbp[0, :_N] + nbp[1, :_N]) * inv[...]
    a = jnp.maximum(jnp.dot(h, w1h[...]) + jnp.dot(nb, w1n[...]) + b1[...],
                    0.0)
    w = jax.nn.sigmoid(jnp.dot(a, w2[...]) + b2[...])
    h2 = h + w * nb
    h2_out[...] = h2
    hs_out[:_N] = h2 * ns[...]
    hs_out[_N:] = jnp.zeros((_TPAD - _N, _H), jnp.float32)


def _conv_body(aggp, nd, w, b, g, beta, h_out):
    agg = (aggp[0, :_N] + aggp[1, :_N]) * nd[...]
    hn = jnp.dot(agg, w[...]) + b[...]
    mean = jnp.mean(hn, axis=0)
    var = jnp.mean((hn - mean) ** 2, axis=0)
    hn = (hn - mean) * lax.rsqrt(var + 1e-5) * g[...] + beta[...]
    h_out[:_N] = jnp.maximum(hn, 0.0)
    h_out[_N:] = jnp.zeros((_TPAD - _N, _H), jnp.float32)


def _conv_res_body(aggp, nd, hprev, w, b, g, beta, h_out):
    agg = (aggp[0, :_N] + aggp[1, :_N]) * nd[...]
    hn = jnp.dot(agg, w[...]) + b[...]
    mean = jnp.mean(hn, axis=0)
    var = jnp.mean((hn - mean) ** 2, axis=0)
    hn = (hn - mean) * lax.rsqrt(var + 1e-5) * g[...] + beta[...]
    h_out[:_N] = jnp.maximum(hn + hprev[...], 0.0)
    h_out[_N:] = jnp.zeros((_TPAD - _N, _H), jnp.float32)


def _conv_fc_body(aggp, nd, hprev, w, b, g, beta, fcw, fcb, out):
    agg = (aggp[0, :_N] + aggp[1, :_N]) * nd[...]
    hn = jnp.dot(agg, w[...]) + b[...]
    mean = jnp.mean(hn, axis=0)
    var = jnp.mean((hn - mean) ** 2, axis=0)
    hn = (hn - mean) * lax.rsqrt(var + 1e-5) * g[...] + beta[...]
    h = jnp.maximum(hn + hprev[...], 0.0)
    out[...] = jnp.dot(h, fcw[...]) + fcb[...]


def _tc(body, out_shape, *args):
    return pl.pallas_call(body, out_shape=out_shape)(*args)


_F32 = jnp.float32
_MAT = jax.ShapeDtypeStruct((_N, _H), _F32)
_MATP = jax.ShapeDtypeStruct((_TPAD, _H), _F32)
_COL = jax.ShapeDtypeStruct((_N, 1), _F32)


def kernel(features, edge_index, enc_W1, enc_b1, enc_W2, enc_b2, enc_W3,
           enc_b3, att_W1, att_b1, att_W2, att_b2, conv_W0, conv_b0,
           conv_W1, conv_b1, conv_W2, conv_b2, conv_W3, conv_b3,
           bn_g0, bn_b0, bn_g1, bn_b1, bn_g2, bn_b2, bn_g3, bn_b3,
           fc_W, fc_b):
    src_flat = edge_index[0]
    dst_flat = edge_index[1]
    src3 = jnp.concatenate(
        [src_flat, _N + (jnp.arange(_EPAD, dtype=jnp.int32) % (_TPAD - _N))]
    ).reshape(_NW, _CH, _K)
    src3 = jnp.concatenate(
        [src3, jnp.full((_NW, _CHS - _CH, _K), _N, jnp.int32)], axis=1)
    dst3 = jnp.concatenate(
        [dst_flat, jnp.arange(_EPAD, dtype=jnp.int32) % _NPAD]
    ).reshape(_NW, _CH, _K)
    dst3 = jnp.concatenate(
        [dst3, jnp.zeros((_NW, _CHS - _CH, _K), jnp.int32)], axis=1)

    odp_raw, idp_raw = _degrees_sc(src_flat, dst_flat)
    odp = odp_raw.reshape(_NC, _NPAD)[:, :_N, None]
    idp = idp_raw.reshape(_NC, _NPAD)[:, :_N, None]

    eb1 = enc_b1.reshape(1, -1)
    eb2 = enc_b2.reshape(1, -1)
    eb3 = enc_b3.reshape(1, -1)
    ab1 = att_b1.reshape(1, -1)
    ab2 = att_b2.reshape(1, -1)
    w1h = att_W1[:_H]
    w1n = att_W1[_H:]
    fcb = fc_b.reshape(1, -1)

    henc = _tc(_enc_body, _MAT,
               features, enc_W1, eb1, enc_W2, eb2, enc_W3, eb3)
    hs, nsc, ndc, invc = _tc(
        _norm_body, (_MATP, _COL, _COL, _COL), henc, odp, idp)

    layer_params = [(conv_W0, conv_b0.reshape(1, -1), bn_g0.reshape(1, -1),
                     bn_b0.reshape(1, -1)),
                    (conv_W1, conv_b1.reshape(1, -1), bn_g1.reshape(1, -1),
                     bn_b1.reshape(1, -1)),
                    (conv_W2, conv_b2.reshape(1, -1), bn_g2.reshape(1, -1),
                     bn_b2.reshape(1, -1)),
                    (conv_W3, conv_b3.reshape(1, -1), bn_g3.reshape(1, -1),
                     bn_b3.reshape(1, -1))]

    aggp = _seg_sum_sc(hs, src3, dst3)
    w0, b0, g0, be0 = layer_params[0]
    h = _tc(_conv_body, _MATP, aggp, ndc, w0, b0, g0, be0)

    for i in (1, 2, 3):
        wi, bi, gi, bei = layer_params[i]
        nbp = _seg_sum_sc(h, src3, dst3)
        h2, hs = _tc(_att_body, (_MAT, _MATP),
                     h, nbp, invc, nsc, w1h, w1n, ab1, att_W2, ab2)
        aggp = _seg_sum_sc(hs, src3, dst3)
        if i < 3:
            h = _tc(_conv_res_body, _MATP, aggp, ndc, h2, wi, bi, gi, bei)
        else:
            out = _tc(_conv_fc_body,
                      jax.ShapeDtypeStruct((_N, fc_W.shape[1]), _F32),
                      aggp, ndc, h2, wi, bi, gi, bei, fc_W, fcb)
    return out


# speedup vs baseline: 1.0017x; 1.0017x over previous
import functools

import jax
import jax.numpy as jnp
from jax import lax
from jax.experimental import pallas as pl
from jax.experimental.pallas import tpu as pltpu
from jax.experimental.pallas import tpu_sc as plsc

_N = 10000
_E = 320000
_D = 128
_H = 128
_NC = 2
_NS = 16
_NW = _NC * _NS
_EPT = _E // _NW
_K = 128
_CH = 79
_CHS = 80
_EPAD = _NW * _CH * _K - _E
_NPAD = 80 * 128
_RPT = _NPAD // _NS
_TPAD = _N + 16

_mesh = plsc.VectorSubcoreMesh(core_axis_name="c", subcore_axis_name="s")
_sc_params = pltpu.CompilerParams(needs_layout_passes=False)


@functools.partial(
    pl.kernel,
    out_type=jax.ShapeDtypeStruct((_NC, _NPAD, _H), jnp.float32),
    mesh=_mesh,
    scratch_types=[
        pltpu.VMEM((_CHS, _K), jnp.int32),
        pltpu.VMEM((2, _K), jnp.int32),
        pltpu.VMEM((_K, _H), jnp.float32),
        pltpu.VMEM((_K, _H), jnp.float32),
        pltpu.VMEM_SHARED((_NPAD, _H), jnp.float32),
        pltpu.SemaphoreType.DMA,
        pltpu.SemaphoreType.DMA,
        pltpu.SemaphoreType.DMA,
        pltpu.SemaphoreType.DMA,
    ],
    compiler_params=_sc_params,
)
def _seg_sum_sc(table_hbm, src_hbm, dst_hbm, out_hbm, srcv, dbuf, rb0, rb1,
                acc, gsem0, gsem1, dsem0, dsem1):
    cid = lax.axis_index("c")
    sid = lax.axis_index("s")
    wid = sid * _NC + cid

    pltpu.sync_copy(src_hbm.at[wid], srcv)
    pltpu.async_copy(table_hbm.at[srcv.at[0]], rb0, gsem0)
    pltpu.async_copy(dst_hbm.at[wid, 0], dbuf.at[0], dsem0)

    zeros16 = jnp.zeros((16,), jnp.float32)

    def _zfill(i, carry):
        for u in range(8):
            rb1[i, pl.ds(u * 16, 16)] = zeros16
        return carry

    lax.fori_loop(0, _K, _zfill, 0)
    base = sid * _RPT
    for r in range(_RPT // _K):
        pltpu.async_copy(rb1, acc.at[pl.ds(base + r * _K, _K)], gsem1)
    for r in range(_RPT // _K):
        pltpu.make_async_copy(rb1, acc.at[pl.ds(base + r * _K, _K)],
                              gsem1).wait()
    plsc.subcore_barrier()

    def _body(jj, carry):
        j0 = jj * 2
        pltpu.make_async_copy(table_hbm.at[srcv.at[j0]], rb0, gsem0).wait()
        pltpu.make_async_copy(dst_hbm.at[wid, j0], dbuf.at[0], dsem0).wait()
        pltpu.async_copy(table_hbm.at[srcv.at[j0 + 1]], rb1, gsem1)
        pltpu.async_copy(dst_hbm.at[wid, j0 + 1], dbuf.at[1], dsem1)
        pltpu.sync_copy(rb0, acc.at[dbuf.at[0]], add=True)
        pltpu.make_async_copy(table_hbm.at[srcv.at[j0 + 1]], rb1, gsem1).wait()
        pltpu.make_async_copy(dst_hbm.at[wid, j0 + 1], dbuf.at[1], dsem1).wait()

        @pl.when(jj < _CH // 2 - 1)
        def _():
            pltpu.async_copy(table_hbm.at[srcv.at[j0 + 2]], rb0, gsem0)
            pltpu.async_copy(dst_hbm.at[wid, j0 + 2], dbuf.at[0], dsem0)

        pltpu.sync_copy(rb1, acc.at[dbuf.at[1]], add=True)
        return carry

    lax.fori_loop(0, _CH // 2, _body, 0)
    if _CH % 2:
        pltpu.sync_copy(dst_hbm.at[wid, _CH - 1], dbuf.at[0])
        pltpu.sync_copy(table_hbm.at[srcv.at[_CH - 1]], rb0)
        pltpu.sync_copy(rb0, acc.at[dbuf.at[0]], add=True)

    plsc.subcore_barrier()
    pltpu.sync_copy(acc.at[pl.ds(sid * _RPT, _RPT)],
                    out_hbm.at[cid, pl.ds(sid * _RPT, _RPT)])


@functools.partial(
    pl.kernel,
    out_type=(jax.ShapeDtypeStruct((_NC, 80, 128), jnp.float32),
              jax.ShapeDtypeStruct((_NC, 80, 128), jnp.float32)),
    mesh=_mesh,
    scratch_types=[
        pltpu.VMEM((_EPT,), jnp.int32),
        pltpu.VMEM((_NPAD,), jnp.float32),
        pltpu.VMEM((_NPAD,), jnp.float32),
        pltpu.VMEM((80, 128), jnp.float32),
        pltpu.VMEM((80, 128), jnp.float32),
        pltpu.VMEM((1, 80), jnp.int32),
        pltpu.VMEM_SHARED((80, 128), jnp.float32),
        pltpu.VMEM_SHARED((80, 128), jnp.float32),
    ],
    compiler_params=_sc_params,
)
def _degrees_sc(src_hbm, dst_hbm, outs_hbm, outd_hbm,
                idxv, h1s, h1d, hs, hd, rows_i, shs, shd):
    cid = lax.axis_index("c")
    sid = lax.axis_index("s")
    wid = sid * _NC + cid

    zeros16 = jnp.zeros((16,), jnp.float32)
    ones16 = jnp.ones((16,), jnp.float32)

    def _zfill(i, carry):
        hs[i >> 3, pl.ds((i & 7) * 16, 16)] = zeros16
        hd[i >> 3, pl.ds((i & 7) * 16, 16)] = zeros16
        h1s[pl.ds(i * 16, 16)] = zeros16
        h1d[pl.ds(i * 16, 16)] = zeros16
        return carry

    lax.fori_loop(0, 80 * 8, _zfill, 0)
    for r in range(5):
        rows_i[0, pl.ds(r * 16, 16)] = lax.iota(jnp.int32, 16) + r * 16

    @pl.when(sid < 10)
    def _():
        pltpu.sync_copy(hs.at[pl.ds(sid * 8, 8)], shs.at[pl.ds(sid * 8, 8)])
        pltpu.sync_copy(hd.at[pl.ds(sid * 8, 8)], shd.at[pl.ds(sid * 8, 8)])

    plsc.subcore_barrier()

    pltpu.sync_copy(src_hbm.at[pl.ds(wid * _EPT, _EPT)], idxv)

    def _hist_s(i, carry):
        for u in range(5):
            v = idxv[pl.ds(i * 80 + u * 16, 16)]
            plsc.addupdate_scatter(h1s, [v], ones16)
        return carry

    lax.fori_loop(0, _EPT // 80, _hist_s, 0)

    pltpu.sync_copy(dst_hbm.at[pl.ds(wid * _EPT, _EPT)], idxv)

    def _hist_d(i, carry):
        for u in range(5):
            v = idxv[pl.ds(i * 80 + u * 16, 16)]
            plsc.addupdate_scatter(h1d, [v], ones16)
        return carry

    lax.fori_loop(0, _EPT // 80, _hist_d, 0)

    def _repack(i, carry):
        hs[i >> 3, pl.ds((i & 7) * 16, 16)] = h1s[pl.ds(i * 16, 16)]
        hd[i >> 3, pl.ds((i & 7) * 16, 16)] = h1d[pl.ds(i * 16, 16)]
        return carry

    lax.fori_loop(0, 80 * 8, _repack, 0)

    pltpu.sync_copy(hs, shs.at[rows_i.at[0]], add=True)
    pltpu.sync_copy(hd, shd.at[rows_i.at[0]], add=True)
    plsc.subcore_barrier()

    @pl.when(sid == 0)
    def _():
        pltpu.sync_copy(shs, outs_hbm.at[cid])
        pltpu.sync_copy(shd, outd_hbm.at[cid])


def _prep_body(feat, odp, idp, w1, b1, w2, b2, w3, b3,
               hs_out, ns_out, nd_out, inv_out):
    od = jnp.maximum(odp[0] + odp[1], 1.0)
    indeg = jnp.maximum(idp[0] + idp[1], 1.0)
    ns = lax.rsqrt(od)
    nd = lax.rsqrt(indeg)
    ns_out[...] = ns
    nd_out[...] = nd
    inv_out[...] = 1.0 / indeg
    x = feat[...]
    h = jnp.maximum(jnp.dot(x, w1[...]) + b1[...], 0.0)
    h = jnp.maximum(jnp.dot(h, w2[...]) + b2[...], 0.0)
    h = jnp.dot(h, w3[...]) + b3[...]
    hs_out[:_N] = h * ns
    hs_out[_N:] = jnp.zeros((_TPAD - _N, _H), jnp.float32)


def _att_body(h_ref, nbp, inv, ns, w1h, w1n, b1, w2, b2, h2_out, hs_out):
    h = h_ref[:_N]
    nb = (nbp[0, :_N] + nbp[1, :_N]) * inv[...]
    a = jnp.maximum(jnp.dot(h, w1h[...]) + jnp.dot(nb, w1n[...]) + b1[...],
                    0.0)
    w = jax.nn.sigmoid(jnp.dot(a, w2[...]) + b2[...])
    h2 = h + w * nb
    h2_out[...] = h2
    hs_out[:_N] = h2 * ns[...]
    hs_out[_N:] = jnp.zeros((_TPAD - _N, _H), jnp.float32)


def _conv_body(aggp, nd, w, b, g, beta, h_out):
    agg = (aggp[0, :_N] + aggp[1, :_N]) * nd[...]
    hn = jnp.dot(agg, w[...]) + b[...]
    mean = jnp.mean(hn, axis=0)
    var = jnp.mean((hn - mean) ** 2, axis=0)
    hn = (hn - mean) * lax.rsqrt(var + 1e-5) * g[...] + beta[...]
    h_out[:_N] = jnp.maximum(hn, 0.0)
    h_out[_N:] = jnp.zeros((_TPAD - _N, _H), jnp.float32)


def _conv_res_body(aggp, nd, hprev, w, b, g, beta, h_out):
    agg = (aggp[0, :_N] + aggp[1, :_N]) * nd[...]
    hn = jnp.dot(agg, w[...]) + b[...]
    mean = jnp.mean(hn, axis=0)
    var = jnp.mean((hn - mean) ** 2, axis=0)
    hn = (hn - mean) * lax.rsqrt(var + 1e-5) * g[...] + beta[...]
    h_out[:_N] = jnp.maximum(hn + hprev[...], 0.0)
    h_out[_N:] = jnp.zeros((_TPAD - _N, _H), jnp.float32)


def _conv_fc_body(aggp, nd, hprev, w, b, g, beta, fcw, fcb, out):
    agg = (aggp[0, :_N] + aggp[1, :_N]) * nd[...]
    hn = jnp.dot(agg, w[...]) + b[...]
    mean = jnp.mean(hn, axis=0)
    var = jnp.mean((hn - mean) ** 2, axis=0)
    hn = (hn - mean) * lax.rsqrt(var + 1e-5) * g[...] + beta[...]
    h = jnp.maximum(hn + hprev[...], 0.0)
    out[...] = jnp.dot(h, fcw[...]) + fcb[...]


def _tc(body, out_shape, *args):
    return pl.pallas_call(body, out_shape=out_shape)(*args)


_F32 = jnp.float32
_MAT = jax.ShapeDtypeStruct((_N, _H), _F32)
_MATP = jax.ShapeDtypeStruct((_TPAD, _H), _F32)
_COL = jax.ShapeDtypeStruct((_N, 1), _F32)


def kernel(features, edge_index, enc_W1, enc_b1, enc_W2, enc_b2, enc_W3,
           enc_b3, att_W1, att_b1, att_W2, att_b2, conv_W0, conv_b0,
           conv_W1, conv_b1, conv_W2, conv_b2, conv_W3, conv_b3,
           bn_g0, bn_b0, bn_g1, bn_b1, bn_g2, bn_b2, bn_g3, bn_b3,
           fc_W, fc_b):
    src_flat = edge_index[0]
    dst_flat = edge_index[1]
    src3 = jnp.concatenate(
        [src_flat, _N + (jnp.arange(_EPAD, dtype=jnp.int32) % (_TPAD - _N))]
    ).reshape(_NW, _CH, _K)
    src3 = jnp.concatenate(
        [src3, jnp.full((_NW, _CHS - _CH, _K), _N, jnp.int32)], axis=1)
    dst3 = jnp.concatenate(
        [dst_flat, jnp.arange(_EPAD, dtype=jnp.int32) % _NPAD]
    ).reshape(_NW, _CH, _K)
    dst3 = jnp.concatenate(
        [dst3, jnp.zeros((_NW, _CHS - _CH, _K), jnp.int32)], axis=1)

    odp_raw, idp_raw = _degrees_sc(src_flat, dst_flat)
    odp = odp_raw.reshape(_NC, _NPAD)[:, :_N, None]
    idp = idp_raw.reshape(_NC, _NPAD)[:, :_N, None]

    eb1 = enc_b1.reshape(1, -1)
    eb2 = enc_b2.reshape(1, -1)
    eb3 = enc_b3.reshape(1, -1)
    ab1 = att_b1.reshape(1, -1)
    ab2 = att_b2.reshape(1, -1)
    w1h = att_W1[:_H]
    w1n = att_W1[_H:]
    fcb = fc_b.reshape(1, -1)

    hs, nsc, ndc, invc = _tc(
        _prep_body, (_MATP, _COL, _COL, _COL),
        features, odp, idp, enc_W1, eb1, enc_W2, eb2, enc_W3, eb3)

    layer_params = [(conv_W0, conv_b0.reshape(1, -1), bn_g0.reshape(1, -1),
                     bn_b0.reshape(1, -1)),
                    (conv_W1, conv_b1.reshape(1, -1), bn_g1.reshape(1, -1),
                     bn_b1.reshape(1, -1)),
                    (conv_W2, conv_b2.reshape(1, -1), bn_g2.reshape(1, -1),
                     bn_b2.reshape(1, -1)),
                    (conv_W3, conv_b3.reshape(1, -1), bn_g3.reshape(1, -1),
                     bn_b3.reshape(1, -1))]

    aggp = _seg_sum_sc(hs, src3, dst3)
    w0, b0, g0, be0 = layer_params[0]
    h = _tc(_conv_body, _MATP, aggp, ndc, w0, b0, g0, be0)

    for i in (1, 2, 3):
        wi, bi, gi, bei = layer_params[i]
        nbp = _seg_sum_sc(h, src3, dst3)
        h2, hs = _tc(_att_body, (_MAT, _MATP),
                     h, nbp, invc, nsc, w1h, w1n, ab1, att_W2, ab2)
        aggp = _seg_sum_sc(hs, src3, dst3)
        if i < 3:
            h = _tc(_conv_res_body, _MATP, aggp, ndc, h2, wi, bi, gi, bei)
        else:
            out = _tc(_conv_fc_body,
                      jax.ShapeDtypeStruct((_N, fc_W.shape[1]), _F32),
                      aggp, ndc, h2, wi, bi, gi, bei, fc_W, fcb)
    return out
